# Initial kernel scaffold; baseline (speedup 1.0000x reference)
import jax
import jax.numpy as jnp
from jax.experimental import pallas as pl


def kernel(x, edge_index, edge_attr, W_ne, b_ne, W_ee, b_ee, Wz, bz, LzW, Lzb, Wr, br, LrW, Lrb, Wh, bh, LhW, Lhb, W_out, b_out):
    raise NotImplementedError("write your pallas kernel here")


# speedup vs baseline: 34.2761x; 34.2761x over previous
import functools

import jax
import jax.numpy as jnp
from jax import lax
from jax.experimental import pallas as pl
from jax.experimental.pallas import tpu as pltpu
from jax.experimental.pallas import tpu_sc as plsc

N = 10000
NP = 10240
E = 320000
D_IN = 128
D_EDGE = 16
HID = 32

NC = 2
NS = 16
NW = NC * NS
E_PER_W = E // NW
CHUNK = 2000
NCH = E_PER_W // CHUNK
SLICE = NP // NS

_mesh = plsc.VectorSubcoreMesh(core_axis_name="c", subcore_axis_name="s")


def _fill_1d(buf, n, val):
    def body(i, carry):
        buf[pl.ds(i * 16, 16)] = jnp.full((16,), val, jnp.float32)
        return carry
    lax.fori_loop(0, n // 16, body, 0)


@functools.partial(
    pl.kernel,
    mesh=_mesh,
    out_type=jax.ShapeDtypeStruct((NC, NP), jnp.float32),
    scratch_types=[
        pltpu.VMEM((CHUNK,), jnp.int32),
        pltpu.VMEM((CHUNK,), jnp.float32),
        pltpu.VMEM((SLICE,), jnp.float32),
        pltpu.VMEM_SHARED((NP,), jnp.float32),
    ],
)
def _deg_sc(col_hbm, deg_hbm, idx_v, ones_v, buf_v, sh_deg):
    c = lax.axis_index("c")
    s = lax.axis_index("s")
    wid = s * NC + c
    _fill_1d(ones_v, CHUNK, 1.0)
    _fill_1d(buf_v, SLICE, 0.0)
    pltpu.sync_copy(buf_v, sh_deg.at[pl.ds(s * SLICE, SLICE)])
    plsc.subcore_barrier()
    for k in range(NCH):
        base = wid * E_PER_W + k * CHUNK
        pltpu.sync_copy(col_hbm.at[pl.ds(base, CHUNK)], idx_v)
        pltpu.sync_copy(ones_v, sh_deg.at[idx_v], add=True)
    plsc.subcore_barrier()
    pltpu.sync_copy(sh_deg.at[pl.ds(s * SLICE, SLICE)], buf_v)
    pltpu.sync_copy(buf_v, deg_hbm.at[c, pl.ds(s * SLICE, SLICE)])


@functools.partial(
    pl.kernel,
    mesh=_mesh,
    compiler_params=pltpu.CompilerParams(use_tc_tiling_on_sc=False),
    out_type=jax.ShapeDtypeStruct((NC, NP, HID), jnp.float32),
    scratch_types=[
        pltpu.VMEM((CHUNK,), jnp.int32),
        pltpu.VMEM((CHUNK,), jnp.int32),
        pltpu.VMEM((CHUNK, HID), jnp.float32),
        pltpu.VMEM((SLICE, HID), jnp.float32),
        pltpu.VMEM_SHARED((NP, HID), jnp.float32),
        pltpu.SemaphoreType.DMA,
    ],
)
def _scatter_sc(y_hbm, row_hbm, col_hbm, s_hbm,
                ridx_v, cidx_v, rows_v, buf_v, sh_s, sem):
    c = lax.axis_index("c")
    s = lax.axis_index("s")
    wid = s * NC + c

    def zbody(i, carry):
        buf_v[i, pl.ds(0, 16)] = jnp.zeros((16,), jnp.float32)
        buf_v[i, pl.ds(16, 16)] = jnp.zeros((16,), jnp.float32)
        return carry
    lax.fori_loop(0, SLICE, zbody, 0)
    pltpu.sync_copy(buf_v, sh_s.at[pl.ds(s * SLICE, SLICE), :])
    plsc.subcore_barrier()
    for k in range(NCH):
        base = wid * E_PER_W + k * CHUNK
        pltpu.sync_copy(row_hbm.at[pl.ds(base, CHUNK)], ridx_v)
        pltpu.sync_copy(col_hbm.at[pl.ds(base, CHUNK)], cidx_v)
        pltpu.async_copy(y_hbm.at[ridx_v], rows_v, sem).wait()
        pltpu.sync_copy(rows_v, sh_s.at[cidx_v], add=True)
    plsc.subcore_barrier()
    pltpu.sync_copy(sh_s.at[pl.ds(s * SLICE, SLICE), :], buf_v)
    pltpu.sync_copy(buf_v, s_hbm.at[c, pl.ds(s * SLICE, SLICE), :])


@functools.partial(
    pl.kernel,
    mesh=_mesh,
    compiler_params=pltpu.CompilerParams(needs_layout_passes=False),
    out_type=jax.ShapeDtypeStruct((E,), jnp.float32),
    scratch_types=[
        pltpu.VMEM((NP,), jnp.float32),
        pltpu.VMEM((NP,), jnp.float32),
        pltpu.VMEM((CHUNK,), jnp.int32),
        pltpu.VMEM((CHUNK,), jnp.int32),
        pltpu.VMEM((CHUNK,), jnp.float32),
        pltpu.VMEM((CHUNK,), jnp.float32),
    ],
)
def _edgeout_sc(u_hbm, v_hbm, eb_hbm, row_hbm, col_hbm, out_hbm,
                u_v, v_v, ridx_v, cidx_v, eb_v, o_v):
    c = lax.axis_index("c")
    s = lax.axis_index("s")
    wid = s * NC + c
    pltpu.sync_copy(u_hbm, u_v)
    pltpu.sync_copy(v_hbm, v_v)
    for k in range(NCH):
        base = wid * E_PER_W + k * CHUNK
        pltpu.sync_copy(row_hbm.at[pl.ds(base, CHUNK)], ridx_v)
        pltpu.sync_copy(col_hbm.at[pl.ds(base, CHUNK)], cidx_v)
        pltpu.sync_copy(eb_hbm.at[pl.ds(base, CHUNK)], eb_v)

        def body(j, carry):
            r = ridx_v[pl.ds(j * 16, 16)]
            cc = cidx_v[pl.ds(j * 16, 16)]
            g = (plsc.load_gather(u_v, [r])
                 + plsc.load_gather(v_v, [cc])
                 + eb_v[pl.ds(j * 16, 16)])
            o_v[pl.ds(j * 16, 16)] = g
            return carry
        lax.fori_loop(0, CHUNK // 16, body, 0)
        pltpu.sync_copy(o_v, out_hbm.at[pl.ds(base, CHUNK)])


def _enc_body(x_ref, w_ref, b_ref, degs_ref, y_ref, dinv_ref):
    xw = jnp.dot(x_ref[...], w_ref[...], preferred_element_type=jnp.float32)
    xe = jnp.maximum(xw + b_ref[...], 0.0)
    deg = degs_ref[:, 0:1] + degs_ref[:, 1:2] + 1.0
    dinv = lax.rsqrt(deg)
    y_ref[...] = xe * dinv
    dinv_ref[...] = dinv


def _enc_tc(x_pad, W_ne, b_ne2, degs_t):
    return pl.pallas_call(
        _enc_body,
        out_shape=[
            jax.ShapeDtypeStruct((NP, HID), jnp.float32),
            jax.ShapeDtypeStruct((NP, 1), jnp.float32),
        ],
    )(x_pad, W_ne, b_ne2, degs_t)


_BE = 6400


def _eb_body(ea_ref, wee_ref, bee_ref, wout_ref, bout_ref, eb_ref):
    t = jnp.dot(ea_ref[...], wee_ref[...], preferred_element_type=jnp.float32)
    t = jnp.maximum(t + bee_ref[...], 0.0)
    w3 = wout_ref[2 * HID:3 * HID, :]
    eb_ref[...] = (jnp.dot(t, w3, preferred_element_type=jnp.float32)
                   + bout_ref[...])


def _eb_tc(edge_attr, W_ee, b_ee2, W_out, b_out2):
    return pl.pallas_call(
        _eb_body,
        grid=(E // _BE,),
        in_specs=[
            pl.BlockSpec((_BE, D_EDGE), lambda i: (i, 0)),
            pl.BlockSpec((D_EDGE, HID), lambda i: (0, 0)),
            pl.BlockSpec((1, HID), lambda i: (0, 0)),
            pl.BlockSpec((3 * HID, 1), lambda i: (0, 0)),
            pl.BlockSpec((1, 1), lambda i: (0, 0)),
        ],
        out_specs=pl.BlockSpec((_BE, 1), lambda i: (i, 0)),
        out_shape=jax.ShapeDtypeStruct((E, 1), jnp.float32),
    )(edge_attr, W_ee, b_ee2, W_out, b_out2)


def _huv_body(s_ref, y_ref, dinv_ref, wz_ref, lzw_ref, lzb_ref, bz_ref,
              wh_ref, lhw_ref, lhb_ref, bh_ref, wout_ref, u_ref, v_ref):
    agg = (s_ref[0] + s_ref[1] + y_ref[...]) * dinv_ref[...]
    lzw = lzw_ref[0:HID, :]
    lhw = lhw_ref[0:HID, :]
    mz = jnp.dot(wz_ref[...], lzw, preferred_element_type=jnp.float32)
    cz = jnp.dot(bz_ref[...], lzw, preferred_element_type=jnp.float32) \
        + lzb_ref[...]
    mh = jnp.dot(wh_ref[...], lhw, preferred_element_type=jnp.float32)
    ch = jnp.dot(bh_ref[...], lhw, preferred_element_type=jnp.float32) \
        + lhb_ref[...]
    z = jax.nn.sigmoid(
        jnp.dot(agg, mz, preferred_element_type=jnp.float32) + cz)
    ht = jnp.tanh(jnp.dot(agg, mh, preferred_element_type=jnp.float32) + ch)
    h = (1.0 - z) * ht
    u_ref[...] = jnp.dot(h, wout_ref[0:HID, :],
                         preferred_element_type=jnp.float32)
    v_ref[...] = jnp.dot(h, wout_ref[HID:2 * HID, :],
                         preferred_element_type=jnp.float32)


def _huv_tc(S, y, dinv, Wz, LzW, Lzb2, bz2, Wh, LhW, Lhb2, bh2, W_out):
    return pl.pallas_call(
        _huv_body,
        out_shape=[
            jax.ShapeDtypeStruct((NP, 1), jnp.float32),
            jax.ShapeDtypeStruct((NP, 1), jnp.float32),
        ],
    )(S, y, dinv, Wz, LzW, Lzb2, bz2, Wh, LhW, Lhb2, bh2, W_out)


def kernel(x, edge_index, edge_attr, W_ne, b_ne, W_ee, b_ee, Wz, bz, LzW,
           Lzb, Wr, br, LrW, Lrb, Wh, bh, LhW, Lhb, W_out, b_out):
    ei = edge_index.astype(jnp.int32)
    row = ei[0]
    col = ei[1]
    x_pad = jnp.pad(x, ((0, NP - N), (0, 0)))

    degs = _deg_sc(col)
    y, dinv = _enc_tc(x_pad, W_ne, b_ne.reshape(1, HID), degs.T)
    S = _scatter_sc(y, row, col)
    u2, v2 = _huv_tc(S, y, dinv, Wz, LzW, Lzb.reshape(1, HID),
                     bz.reshape(1, HID), Wh, LhW, Lhb.reshape(1, HID),
                     bh.reshape(1, HID), W_out)
    eb = _eb_tc(edge_attr, W_ee, b_ee.reshape(1, HID), W_out,
                b_out.reshape(1, 1))
    out = _edgeout_sc(u2.reshape(NP), v2.reshape(NP), eb.reshape(E),
                      row, col)
    return out.reshape(E, 1)
